# Initial kernel scaffold; baseline (speedup 1.0000x reference)
import jax
import jax.numpy as jnp
from jax.experimental import pallas as pl


def kernel(x, edge_index, W1, b1, W2, b2):
    raise NotImplementedError("write your pallas kernel here")


# speedup vs baseline: 9.3198x; 9.3198x over previous
import dataclasses
import functools

import jax
import jax.numpy as jnp
from jax import lax
from jax.experimental import pallas as pl
from jax.experimental.pallas import tpu as pltpu
from jax.experimental.pallas import tpu_sc as plsc

N = 10000
NP = 10240
E = 160000
EP = 163840
PAD = N
IN_F = 256
HID = 512
OUT_F = 128
NSC = 2
NTI = 16
ROWS_PER_TILE = NP // NTI
EDGES_PER_TILE = EP // NTI
NB = EDGES_PER_TILE // 128
CH = 16
NCH = NB // CH
DEG_EPT = EP // (NSC * NTI)
BN = 256

_VMESH = plsc.VectorSubcoreMesh(core_axis_name="core", subcore_axis_name="subcore")

_CP = pltpu.CompilerParams()
if "needs_layout_passes" in pltpu.CompilerParams.__dataclass_fields__:
    _CP = dataclasses.replace(_CP, needs_layout_passes=False)


@functools.partial(
    pl.kernel,
    out_type=jax.ShapeDtypeStruct((NSC * NTI * NP,), jnp.float32),
    mesh=_VMESH,
    scratch_types=[
        pltpu.VMEM((NP,), jnp.float32),
        pltpu.VMEM((DEG_EPT,), jnp.int32),
    ],
    compiler_params=_CP,
)
def _deg_kernel(dst_hbm, out_hbm, deg_v, idx_v):
    c = lax.axis_index("core")
    s = lax.axis_index("subcore")
    w = s * NSC + c
    ebase = pl.multiple_of(w * DEG_EPT, DEG_EPT)
    pltpu.sync_copy(dst_hbm.at[pl.ds(ebase, DEG_EPT)], idx_v)

    z16 = jnp.zeros((16,), jnp.float32)

    @pl.loop(0, NP, step=16)
    def _(i):
        deg_v[pl.ds(i, 16)] = z16

    ones = jnp.ones((16,), jnp.float32)

    @pl.loop(0, DEG_EPT, step=16)
    def _(j):
        plsc.addupdate_scatter(deg_v, [idx_v[pl.ds(j, 16)]], ones)

    obase = pl.multiple_of(w * NP, NP)
    pltpu.sync_copy(deg_v, out_hbm.at[pl.ds(obase, NP)])


def _make_prop(feature_split, nb, ch):
    nch = nb // ch
    @functools.partial(
        pl.kernel,
        out_type=jax.ShapeDtypeStruct((NSC * NP, 128), jnp.float32),
        mesh=_VMESH,
        scratch_types=[
            pltpu.VMEM((ch * 128,), jnp.int32),
            pltpu.VMEM((nb, 128), jnp.int32),
            pltpu.VMEM((128, 128), jnp.float32),
            pltpu.VMEM((128, 128), jnp.float32),
            pltpu.VMEM_SHARED((NP, 128), jnp.float32),
            pltpu.SemaphoreType.DMA,
            pltpu.SemaphoreType.DMA,
        ],
        compiler_params=_CP,
    )
    def prop(u_hbm, src_hbm, dst_hbm, t_hbm, sidx, didx, rows0, rows1, acc,
             sem0, sem1):
        c = lax.axis_index("core")
        s = lax.axis_index("subcore")
        if feature_split:
            tile_ebase = s * (nb * 128)
        else:
            tile_ebase = (s * NSC + c) * (nb * 128)
        rowbase = pl.multiple_of(tile_ebase // 128, 8)
        pltpu.sync_copy(dst_hbm.at[pl.ds(rowbase, nb)], didx)

        z16 = jnp.zeros((16,), jnp.float32)

        @pl.loop(0, 128)
        def _(r):
            @pl.loop(0, 128, step=16)
            def _(j):
                rows0[r, pl.ds(j, 16)] = z16

        @pl.loop(0, ROWS_PER_TILE, step=128)
        def _(r):
            abase = pl.multiple_of(s * ROWS_PER_TILE + r, 128)
            pltpu.sync_copy(rows0, acc.at[pl.ds(abase, 128)])

        plsc.subcore_barrier()

        def start_gather(b, buf, sem):
            pltpu.async_copy(u_hbm.at[sidx.at[pl.ds(b * 128, 128)]], buf, sem)

        def wait_gather(buf, sem):
            pltpu.make_async_copy(
                u_hbm.at[sidx.at[pl.ds(0, 128)]], buf, sem).wait()

        @pl.loop(0, nch)
        def _(k):
            ebase = pl.multiple_of(tile_ebase + k * (ch * 128), 128)
            pltpu.sync_copy(src_hbm.at[pl.ds(ebase, ch * 128)], sidx)

            if feature_split:
                @pl.loop(0, ch * 128, step=16)
                def _(j):
                    sidx[pl.ds(j, 16)] = sidx[pl.ds(j, 16)] * 2 + c

            start_gather(0, rows0, sem0)
            start_gather(1, rows1, sem1)
            dbase = k * ch

            @pl.loop(0, ch - 2, step=2)
            def _(g):
                wait_gather(rows0, sem0)
                pltpu.sync_copy(rows0, acc.at[didx.at[dbase + g]], add=True)
                start_gather(g + 2, rows0, sem0)
                wait_gather(rows1, sem1)
                pltpu.sync_copy(rows1, acc.at[didx.at[dbase + g + 1]], add=True)
                start_gather(g + 3, rows1, sem1)

            wait_gather(rows0, sem0)
            pltpu.sync_copy(rows0, acc.at[didx.at[dbase + ch - 2]], add=True)
            wait_gather(rows1, sem1)
            pltpu.sync_copy(rows1, acc.at[didx.at[dbase + ch - 1]], add=True)

        plsc.subcore_barrier()
        base = pl.multiple_of(s * ROWS_PER_TILE, 128)
        tbase = pl.multiple_of(c * NP + s * ROWS_PER_TILE, 128)
        pltpu.sync_copy(acc.at[pl.ds(base, ROWS_PER_TILE)],
                        t_hbm.at[pl.ds(tbase, ROWS_PER_TILE)])

    return prop


_prop_f = _make_prop(True, NB, CH)
_prop_e = _make_prop(False, EP // (32 * 128), 10)


def _k0_body(deg_ref, x_ref, dinv_ref, u1_ref):
    d = jnp.sum(deg_ref[...], axis=0) + 1.0
    di = lax.rsqrt(d)[:, None]
    dinv_ref[...] = di
    u1_ref[...] = x_ref[...] * di


_k0 = pl.pallas_call(
    _k0_body,
    grid=(NP // BN,),
    in_specs=[
        pl.BlockSpec((NSC * NTI, BN), lambda i: (0, i)),
        pl.BlockSpec((BN, IN_F), lambda i: (i, 0)),
    ],
    out_specs=[
        pl.BlockSpec((BN, 1), lambda i: (i, 0)),
        pl.BlockSpec((BN, IN_F), lambda i: (i, 0)),
    ],
    out_shape=[
        jax.ShapeDtypeStruct((NP, 1), jnp.float32),
        jax.ShapeDtypeStruct((NP, IN_F), jnp.float32),
    ],
)


def _k1_body(t1_ref, u1_ref, dinv_ref, w1_ref, b1_ref, w2_ref, u2_ref):
    di = dinv_ref[...]
    t = jnp.concatenate([t1_ref[0], t1_ref[1]], axis=1)
    p1 = di * (t + u1_ref[...])
    h = jnp.dot(p1, w1_ref[...], preferred_element_type=jnp.float32)
    h = jnp.maximum(h + b1_ref[...], 0.0)
    q2 = jnp.dot(h, w2_ref[...], preferred_element_type=jnp.float32)
    u2_ref[...] = di * q2


_k1 = pl.pallas_call(
    _k1_body,
    grid=(NP // BN,),
    in_specs=[
        pl.BlockSpec((NSC, BN, 128), lambda i: (0, i, 0)),
        pl.BlockSpec((BN, IN_F), lambda i: (i, 0)),
        pl.BlockSpec((BN, 1), lambda i: (i, 0)),
        pl.BlockSpec((IN_F, HID), lambda i: (0, 0)),
        pl.BlockSpec((1, HID), lambda i: (0, 0)),
        pl.BlockSpec((HID, OUT_F), lambda i: (0, 0)),
    ],
    out_specs=pl.BlockSpec((BN, OUT_F), lambda i: (i, 0)),
    out_shape=jax.ShapeDtypeStruct((NP, OUT_F), jnp.float32),
)


def _k2_body(t2_ref, u2_ref, dinv_ref, b2_ref, out_ref):
    di = dinv_ref[...]
    t = t2_ref[0] + t2_ref[1]
    out_ref[...] = di * (t + u2_ref[...]) + b2_ref[...]


_k2 = pl.pallas_call(
    _k2_body,
    grid=(NP // BN,),
    in_specs=[
        pl.BlockSpec((NSC, BN, OUT_F), lambda i: (0, i, 0)),
        pl.BlockSpec((BN, OUT_F), lambda i: (i, 0)),
        pl.BlockSpec((BN, 1), lambda i: (i, 0)),
        pl.BlockSpec((1, OUT_F), lambda i: (0, 0)),
    ],
    out_specs=pl.BlockSpec((BN, OUT_F), lambda i: (i, 0)),
    out_shape=jax.ShapeDtypeStruct((NP, OUT_F), jnp.float32),
)


def kernel(x, edge_index, W1, b1, W2, b2):
    src = edge_index[0].astype(jnp.int32)
    dst = edge_index[1].astype(jnp.int32)
    padl = EP - E
    src_flat = jnp.concatenate([src, jnp.full((padl,), PAD, jnp.int32)])
    dst_flat = jnp.concatenate([dst, jnp.full((padl,), PAD, jnp.int32)])
    dst2d = dst_flat.reshape(EP // 128, 128)
    x_pad = jnp.pad(x, ((0, NP - N), (0, 0)))

    deg_parts = _deg_kernel(dst_flat).reshape(NSC * NTI, NP)
    dinv, u1 = _k0(deg_parts, x_pad)

    t1 = _prop_f(u1.reshape(NSC * NP, 128), src_flat, dst2d)
    u2 = _k1(t1.reshape(NSC, NP, 128), u1, dinv, W1, b1.reshape(1, HID), W2)
    t2 = _prop_e(u2, src_flat, dst2d)
    logits = _k2(t2.reshape(NSC, NP, OUT_F), u2, dinv, b2.reshape(1, OUT_F))
    return logits[:N]


# speedup vs baseline: 20.7335x; 2.2247x over previous
import dataclasses
import functools

import jax
import jax.numpy as jnp
from jax import lax
from jax.experimental import pallas as pl
from jax.experimental.pallas import tpu as pltpu
from jax.experimental.pallas import tpu_sc as plsc

N = 10000
NP = 10240
E = 160000
EP = 163840
PAD = N
IN_F = 256
HID = 512
OUT_F = 128
NSC = 2
NTI = 16
ROWS_PER_TILE = NP // NTI
EDGES_PER_TILE = EP // NTI
NB = EDGES_PER_TILE // 128
CH = 16
NCH = NB // CH
DEG_EPT = EP // (NSC * NTI)
BN = 256

_VMESH = plsc.VectorSubcoreMesh(core_axis_name="core", subcore_axis_name="subcore")

_CP = pltpu.CompilerParams()
if "needs_layout_passes" in pltpu.CompilerParams.__dataclass_fields__:
    _CP = dataclasses.replace(_CP, needs_layout_passes=False)


@functools.partial(
    pl.kernel,
    out_type=jax.ShapeDtypeStruct((NSC * NTI * NP,), jnp.float32),
    mesh=_VMESH,
    scratch_types=[
        pltpu.VMEM((NP,), jnp.float32),
        pltpu.VMEM((DEG_EPT,), jnp.int32),
    ],
    compiler_params=_CP,
)
def _deg_kernel(dst_hbm, out_hbm, deg_v, idx_v):
    c = lax.axis_index("core")
    s = lax.axis_index("subcore")
    w = s * NSC + c
    ebase = pl.multiple_of(w * DEG_EPT, DEG_EPT)
    pltpu.sync_copy(dst_hbm.at[pl.ds(ebase, DEG_EPT)], idx_v)

    z16 = jnp.zeros((16,), jnp.float32)

    @pl.loop(0, NP, step=16)
    def _(i):
        deg_v[pl.ds(i, 16)] = z16

    ones = jnp.ones((16,), jnp.float32)

    @pl.loop(0, DEG_EPT, step=16)
    def _(j):
        plsc.addupdate_scatter(deg_v, [idx_v[pl.ds(j, 16)]], ones)

    obase = pl.multiple_of(w * NP, NP)
    pltpu.sync_copy(deg_v, out_hbm.at[pl.ds(obase, NP)])


def _make_prop(feature_split, nb, ch):
    nch = nb // ch
    @functools.partial(
        pl.kernel,
        out_type=jax.ShapeDtypeStruct((NSC * NP, 128), jnp.float32),
        mesh=_VMESH,
        scratch_types=[
            pltpu.VMEM((ch * 128,), jnp.int32),
            pltpu.VMEM((nb, 128), jnp.int32),
            pltpu.VMEM((128, 128), jnp.float32),
            pltpu.VMEM((128, 128), jnp.float32),
            pltpu.VMEM_SHARED((NP, 128), jnp.float32),
            pltpu.SemaphoreType.DMA,
            pltpu.SemaphoreType.DMA,
        ],
        compiler_params=_CP,
    )
    def prop(u_hbm, src_hbm, dst_hbm, t_hbm, sidx, didx, rows0, rows1, acc,
             sem0, sem1):
        c = lax.axis_index("core")
        s = lax.axis_index("subcore")
        if feature_split:
            tile_ebase = s * (nb * 128)
        else:
            tile_ebase = (s * NSC + c) * (nb * 128)
        rowbase = pl.multiple_of(tile_ebase // 128, 8)
        pltpu.sync_copy(dst_hbm.at[pl.ds(rowbase, nb)], didx)

        z16 = jnp.zeros((16,), jnp.float32)

        @pl.loop(0, 128)
        def _(r):
            @pl.loop(0, 128, step=16)
            def _(j):
                rows0[r, pl.ds(j, 16)] = z16

        @pl.loop(0, ROWS_PER_TILE, step=128)
        def _(r):
            abase = pl.multiple_of(s * ROWS_PER_TILE + r, 128)
            pltpu.sync_copy(rows0, acc.at[pl.ds(abase, 128)])

        plsc.subcore_barrier()

        def start_gather(b, buf, sem):
            pltpu.async_copy(u_hbm.at[sidx.at[pl.ds(b * 128, 128)]], buf, sem)

        def wait_gather(buf, sem):
            pltpu.make_async_copy(
                u_hbm.at[sidx.at[pl.ds(0, 128)]], buf, sem).wait()

        @pl.loop(0, nch)
        def _(k):
            ebase = pl.multiple_of(tile_ebase + k * (ch * 128), 128)
            pltpu.sync_copy(src_hbm.at[pl.ds(ebase, ch * 128)], sidx)

            if feature_split:
                @pl.loop(0, ch * 128, step=16)
                def _(j):
                    sidx[pl.ds(j, 16)] = sidx[pl.ds(j, 16)] * 2 + c

            start_gather(0, rows0, sem0)
            start_gather(1, rows1, sem1)
            dbase = k * ch

            @pl.loop(0, ch - 2, step=2)
            def _(g):
                wait_gather(rows0, sem0)
                pltpu.sync_copy(rows0, acc.at[didx.at[dbase + g]], add=True)
                start_gather(g + 2, rows0, sem0)
                wait_gather(rows1, sem1)
                pltpu.sync_copy(rows1, acc.at[didx.at[dbase + g + 1]], add=True)
                start_gather(g + 3, rows1, sem1)

            wait_gather(rows0, sem0)
            pltpu.sync_copy(rows0, acc.at[didx.at[dbase + ch - 2]], add=True)
            wait_gather(rows1, sem1)
            pltpu.sync_copy(rows1, acc.at[didx.at[dbase + ch - 1]], add=True)

        plsc.subcore_barrier()
        base = pl.multiple_of(s * ROWS_PER_TILE, 128)
        tbase = pl.multiple_of(c * NP + s * ROWS_PER_TILE, 128)
        pltpu.sync_copy(acc.at[pl.ds(base, ROWS_PER_TILE)],
                        t_hbm.at[pl.ds(tbase, ROWS_PER_TILE)])

    return prop


_prop_f = _make_prop(True, NB, CH)
_prop_e = _make_prop(False, EP // (32 * 128), 10)


def _k0_body(deg_ref, x_ref, dinv_ref, u1_ref):
    d = jnp.sum(deg_ref[...], axis=0) + 1.0
    di = lax.rsqrt(d)[:, None]
    dinv_ref[...] = di
    u1_ref[...] = x_ref[...] * di


_k0 = pl.pallas_call(
    _k0_body,
    grid=(NP // BN,),
    in_specs=[
        pl.BlockSpec((NSC * NTI, BN), lambda i: (0, i)),
        pl.BlockSpec((BN, IN_F), lambda i: (i, 0)),
    ],
    out_specs=[
        pl.BlockSpec((BN, 1), lambda i: (i, 0)),
        pl.BlockSpec((BN, IN_F), lambda i: (i, 0)),
    ],
    out_shape=[
        jax.ShapeDtypeStruct((NP, 1), jnp.float32),
        jax.ShapeDtypeStruct((NP, IN_F), jnp.float32),
    ],
)


def _k1_body(t1_ref, u1_ref, dinv_ref, w1_ref, b1_ref, w2_ref, u2_ref):
    di = dinv_ref[...]
    t = jnp.concatenate([t1_ref[0], t1_ref[1]], axis=1)
    p1 = di * (t + u1_ref[...])
    h = jnp.dot(p1, w1_ref[...], preferred_element_type=jnp.float32)
    h = jnp.maximum(h + b1_ref[...], 0.0)
    q2 = jnp.dot(h, w2_ref[...], preferred_element_type=jnp.float32)
    u2_ref[...] = di * q2


_k1 = pl.pallas_call(
    _k1_body,
    grid=(NP // BN,),
    in_specs=[
        pl.BlockSpec((NSC, BN, 128), lambda i: (0, i, 0)),
        pl.BlockSpec((BN, IN_F), lambda i: (i, 0)),
        pl.BlockSpec((BN, 1), lambda i: (i, 0)),
        pl.BlockSpec((IN_F, HID), lambda i: (0, 0)),
        pl.BlockSpec((1, HID), lambda i: (0, 0)),
        pl.BlockSpec((HID, OUT_F), lambda i: (0, 0)),
    ],
    out_specs=pl.BlockSpec((BN, OUT_F), lambda i: (i, 0)),
    out_shape=jax.ShapeDtypeStruct((NP, OUT_F), jnp.float32),
)


def _k2_body(t2_ref, u2_ref, dinv_ref, b2_ref, out_ref):
    di = dinv_ref[...]
    t = t2_ref[0] + t2_ref[1]
    out_ref[...] = di * (t + u2_ref[...]) + b2_ref[...]


_k2 = pl.pallas_call(
    _k2_body,
    grid=(NP // BN,),
    in_specs=[
        pl.BlockSpec((NSC, BN, OUT_F), lambda i: (0, i, 0)),
        pl.BlockSpec((BN, OUT_F), lambda i: (i, 0)),
        pl.BlockSpec((BN, 1), lambda i: (i, 0)),
        pl.BlockSpec((1, OUT_F), lambda i: (0, 0)),
    ],
    out_specs=pl.BlockSpec((BN, OUT_F), lambda i: (i, 0)),
    out_shape=jax.ShapeDtypeStruct((NP, OUT_F), jnp.float32),
)


def kernel(x, edge_index, W1, b1, W2, b2):
    src = edge_index[0].astype(jnp.int32)
    dst = edge_index[1].astype(jnp.int32)
    padl = EP - E
    pad_ids = PAD + jnp.arange(padl, dtype=jnp.int32) % (NP - N)
    src_flat = jnp.concatenate([src, pad_ids])
    dst_flat = jnp.concatenate([dst, pad_ids])
    dst2d = dst_flat.reshape(EP // 128, 128)
    x_pad = jnp.pad(x, ((0, NP - N), (0, 0)))

    deg_parts = _deg_kernel(dst_flat).reshape(NSC * NTI, NP)
    dinv, u1 = _k0(deg_parts, x_pad)

    t1 = _prop_f(u1.reshape(NSC * NP, 128), src_flat, dst2d)
    u2 = _k1(t1.reshape(NSC, NP, 128), u1, dinv, W1, b1.reshape(1, HID), W2)
    t2 = _prop_e(u2, src_flat, dst2d)
    logits = _k2(t2.reshape(NSC, NP, OUT_F), u2, dinv, b2.reshape(1, OUT_F))
    return logits[:N]
